# baseline (speedup 1.0000x reference)
import functools

import jax
import jax.numpy as jnp
from jax.experimental import pallas as pl

TOPK = 64
TOTAL = 256
VOCAB = 100000
VPAD = 100096
ROWS = VPAD // 128
BEAMS = 64
BIG = 1 << 30
NEGINF = float("-inf")


def _beam_topk_kernel(logits_ref, prev_ref, cu_ref, idx_ref):
    a0 = logits_ref[0]
    flat = (jax.lax.broadcasted_iota(jnp.int32, (ROWS, 128), 0) * 128
            + jax.lax.broadcasted_iota(jnp.int32, (ROWS, 128), 1))
    lane = jax.lax.broadcasted_iota(jnp.int32, (1, 128), 1)

    m0 = jnp.max(a0)
    lse = m0 + jnp.log(jnp.sum(jnp.exp(a0 - m0)))

    def body(i, carry):
        a, vacc, iacc = carry
        g = jnp.max(a)
        fpos = jnp.min(jnp.where(a == g, flat, BIG))
        vacc = jnp.where(lane == i, g, vacc)
        iacc = jnp.where(lane == i, fpos, iacc)
        a = jnp.where(flat == fpos, NEGINF, a)
        return a, vacc, iacc

    vinit = jnp.full((1, 128), NEGINF, jnp.float32)
    iinit = jnp.zeros((1, 128), jnp.int32)
    _, vacc, iacc = jax.lax.fori_loop(0, TOPK, body, (a0, vinit, iinit))

    cu_ref[0] = jnp.broadcast_to(vacc - lse + prev_ref[0, 0, 0], (8, 128))
    idx_ref[0] = jnp.broadcast_to(iacc, (8, 128))


def _merge_kernel(cu_ref, idx_ref, tsp_ref, sidx_ref, dt_ref, mi_ref,
                  ni_ref, oid_ref):
    cu = cu_ref[...]
    vidx = idx_ref[...]
    lane = jax.lax.broadcasted_iota(jnp.int32, (BEAMS, 128), 1)
    row = jax.lax.broadcasted_iota(jnp.int32, (BEAMS, 128), 0)
    fidx = jnp.where(lane < TOPK, row * TOPK + lane, BIG)
    val = jnp.where(lane < TOPK, cu, NEGINF)
    pos2 = (jax.lax.broadcasted_iota(jnp.int32, (2, 128), 0) * 128
            + jax.lax.broadcasted_iota(jnp.int32, (2, 128), 1))

    def body_a(i, carry):
        w, tv, ti = carry
        g = jnp.max(w)
        fp = jnp.min(jnp.where(w == g, fidx, BIG))
        tv = jnp.where(pos2 == i, g, tv)
        ti = jnp.where(pos2 == i, fp, ti)
        w = jnp.where(fidx == fp, NEGINF, w)
        return w, tv, ti

    tv0 = jnp.full((2, 128), NEGINF, jnp.float32)
    ti0 = jnp.zeros((2, 128), jnp.int32)
    _, tspv, tspi = jax.lax.fori_loop(0, TOTAL, body_a, (val, tv0, ti0))

    def body_b(p, sidx):
        ip = jnp.max(jnp.where(pos2 == p, tspi, -1))
        rank = jnp.sum((tspi < ip).astype(jnp.int32))
        return jnp.where(pos2 == rank, ip, sidx)

    sidx = jax.lax.fori_loop(0, TOTAL, body_b, jnp.zeros((2, 128), jnp.int32))

    def body_c(r, carry):
        dt, mi = carry
        sv = jnp.max(jnp.where(pos2 == r, sidx, -1))
        par = sv // TOPK
        tok = jnp.max(jnp.where(fidx == sv, vidx, 0))
        cnt = jnp.sum((sidx < par - 1).astype(jnp.int32))
        m = jnp.where(par == 0, -1, cnt) + 1
        dt = jnp.where(pos2 == r, tok, dt)
        mi = jnp.where(pos2 == r, m, mi)
        return dt, mi

    dt0 = jnp.zeros((2, 128), jnp.int32)
    mi0 = jnp.zeros((2, 128), jnp.int32)
    dt, mi = jax.lax.fori_loop(0, TOTAL, body_c, (dt0, mi0))

    lane1 = jax.lax.broadcasted_iota(jnp.int32, (1, 128), 1)

    def body_d(j, ni):
        tci = jnp.max(jnp.where(pos2 == j, tspi, -1))
        tok = jnp.max(jnp.where(fidx == tci, vidx, 0))
        return jnp.where(lane1 == j, tok, ni)

    ni = jax.lax.fori_loop(0, TOPK, body_d, jnp.zeros((1, 128), jnp.int32))

    tsp_ref[...] = tspv
    sidx_ref[...] = sidx
    dt_ref[...] = dt
    mi_ref[...] = mi
    ni_ref[...] = ni
    oid_ref[...] = tspi // TOPK


@jax.jit
def kernel(logits, prev_scores):
    lp = jnp.pad(logits, ((0, 0), (0, VPAD - VOCAB)),
                 constant_values=-jnp.inf).reshape(BEAMS, ROWS, 128)
    prev = jnp.broadcast_to(prev_scores.reshape(BEAMS, 1, 1), (BEAMS, 8, 128))

    cu3, idx3 = pl.pallas_call(
        _beam_topk_kernel,
        grid=(BEAMS,),
        in_specs=[
            pl.BlockSpec((1, ROWS, 128), lambda i: (i, 0, 0)),
            pl.BlockSpec((1, 8, 128), lambda i: (i, 0, 0)),
        ],
        out_specs=[
            pl.BlockSpec((1, 8, 128), lambda i: (i, 0, 0)),
            pl.BlockSpec((1, 8, 128), lambda i: (i, 0, 0)),
        ],
        out_shape=[
            jax.ShapeDtypeStruct((BEAMS, 8, 128), jnp.float32),
            jax.ShapeDtypeStruct((BEAMS, 8, 128), jnp.int32),
        ],
    )(lp, prev)
    cu = cu3[:, 0, :]
    idx = idx3[:, 0, :]

    tsp, sidx, dt, mi, ni, oid = pl.pallas_call(
        _merge_kernel,
        out_shape=[
            jax.ShapeDtypeStruct((2, 128), jnp.float32),
            jax.ShapeDtypeStruct((2, 128), jnp.int32),
            jax.ShapeDtypeStruct((2, 128), jnp.int32),
            jax.ShapeDtypeStruct((2, 128), jnp.int32),
            jax.ShapeDtypeStruct((1, 128), jnp.int32),
            jax.ShapeDtypeStruct((2, 128), jnp.int32),
        ],
    )(cu, idx)

    top_scores_p = tsp.reshape(TOTAL)
    sorted_index = sidx.reshape(TOTAL)
    draft_tokens = dt.reshape(TOTAL)
    mask_index = mi.reshape(TOTAL)
    topk_cs_p = top_scores_p[:TOPK]
    new_input_ids = ni.reshape(128)[:TOPK]
    out_ids = oid.reshape(TOTAL)[:TOPK]
    return (top_scores_p, draft_tokens, mask_index, topk_cs_p,
            new_input_ids, out_ids, sorted_index)


# speedup vs baseline: 1.0192x; 1.0192x over previous
import functools

import jax
import jax.numpy as jnp
from jax.experimental import pallas as pl
from jax.experimental.pallas import tpu as pltpu

TOPK = 64
TOTAL = 256
VOCAB = 100000
CHUNKS = 98
VPAD = CHUNKS * 8 * 128
ROWS = VPAD // 128
BEAMS = 64
BIG = 1 << 30
NEGINF = float("-inf")


def _beam_topk_kernel(logits_ref, prev_ref, cu_ref, idx_ref, a_ref):
    a0 = logits_ref[0]
    a_ref[...] = a0
    lane = jax.lax.broadcasted_iota(jnp.int32, (1, 128), 1)
    ci = jax.lax.broadcasted_iota(jnp.int32, (CHUNKS, 128), 0)
    p8 = (jax.lax.broadcasted_iota(jnp.int32, (8, 128), 0) * 128
          + jax.lax.broadcasted_iota(jnp.int32, (8, 128), 1))

    m0 = jnp.max(a0)
    lse = m0 + jnp.log(jnp.sum(jnp.exp(a0 - m0)))

    def body(i, carry):
        b, vacc, iacc = carry
        g = jnp.max(b)
        wc = jnp.min(jnp.where(b == g, ci, BIG))
        t = a_ref[wc]
        fp8 = jnp.min(jnp.where(t == g, p8, BIG))
        fpos = wc * 1024 + fp8
        vacc = jnp.where(lane == i, g, vacc)
        iacc = jnp.where(lane == i, fpos, iacc)
        t = jnp.where(p8 == fp8, NEGINF, t)
        a_ref[wc] = t
        nb = jnp.max(t, axis=0, keepdims=True)
        b = jnp.where(ci == wc, nb, b)
        return b, vacc, iacc

    b0 = jnp.max(a0, axis=1)
    vinit = jnp.full((1, 128), NEGINF, jnp.float32)
    iinit = jnp.zeros((1, 128), jnp.int32)
    _, vacc, iacc = jax.lax.fori_loop(0, TOPK, body, (b0, vinit, iinit))

    cu_ref[0] = jnp.broadcast_to(vacc - lse + prev_ref[0, 0, 0], (8, 128))
    idx_ref[0] = jnp.broadcast_to(iacc, (8, 128))


def _merge_kernel(cu_ref, idx_ref, tsp_ref, sidx_ref, dt_ref, mi_ref,
                  ni_ref, oid_ref):
    cu = cu_ref[...]
    vidx = idx_ref[...]
    lane = jax.lax.broadcasted_iota(jnp.int32, (BEAMS, 128), 1)
    row = jax.lax.broadcasted_iota(jnp.int32, (BEAMS, 128), 0)
    fidx = jnp.where(lane < TOPK, row * TOPK + lane, BIG)
    val = jnp.where(lane < TOPK, cu, NEGINF)
    pos2 = (jax.lax.broadcasted_iota(jnp.int32, (2, 128), 0) * 128
            + jax.lax.broadcasted_iota(jnp.int32, (2, 128), 1))

    def body_a(i, carry):
        w, tv, ti = carry
        g = jnp.max(w)
        fp = jnp.min(jnp.where(w == g, fidx, BIG))
        tv = jnp.where(pos2 == i, g, tv)
        ti = jnp.where(pos2 == i, fp, ti)
        w = jnp.where(fidx == fp, NEGINF, w)
        return w, tv, ti

    tv0 = jnp.full((2, 128), NEGINF, jnp.float32)
    ti0 = jnp.zeros((2, 128), jnp.int32)
    _, tspv, tspi = jax.lax.fori_loop(0, TOTAL, body_a, (val, tv0, ti0))

    def body_b(p, sidx):
        ip = jnp.max(jnp.where(pos2 == p, tspi, -1))
        rank = jnp.sum((tspi < ip).astype(jnp.int32))
        return jnp.where(pos2 == rank, ip, sidx)

    sidx = jax.lax.fori_loop(0, TOTAL, body_b, jnp.zeros((2, 128), jnp.int32))

    def body_c(r, carry):
        dt, mi = carry
        sv = jnp.max(jnp.where(pos2 == r, sidx, -1))
        par = sv // TOPK
        tok = jnp.max(jnp.where(fidx == sv, vidx, 0))
        cnt = jnp.sum((sidx < par - 1).astype(jnp.int32))
        m = jnp.where(par == 0, -1, cnt) + 1
        dt = jnp.where(pos2 == r, tok, dt)
        mi = jnp.where(pos2 == r, m, mi)
        return dt, mi

    dt0 = jnp.zeros((2, 128), jnp.int32)
    mi0 = jnp.zeros((2, 128), jnp.int32)
    dt, mi = jax.lax.fori_loop(0, TOTAL, body_c, (dt0, mi0))

    lane1 = jax.lax.broadcasted_iota(jnp.int32, (1, 128), 1)

    def body_d(j, ni):
        tci = jnp.max(jnp.where(pos2 == j, tspi, -1))
        tok = jnp.max(jnp.where(fidx == tci, vidx, 0))
        return jnp.where(lane1 == j, tok, ni)

    ni = jax.lax.fori_loop(0, TOPK, body_d, jnp.zeros((1, 128), jnp.int32))

    tsp_ref[...] = tspv
    sidx_ref[...] = sidx
    dt_ref[...] = dt
    mi_ref[...] = mi
    ni_ref[...] = ni
    oid_ref[...] = tspi // TOPK


@jax.jit
def kernel(logits, prev_scores):
    lp = jnp.pad(logits, ((0, 0), (0, VPAD - VOCAB)),
                 constant_values=-jnp.inf).reshape(BEAMS, CHUNKS, 8, 128)
    prev = jnp.broadcast_to(prev_scores.reshape(BEAMS, 1, 1), (BEAMS, 8, 128))

    cu3, idx3 = pl.pallas_call(
        _beam_topk_kernel,
        grid=(BEAMS,),
        in_specs=[
            pl.BlockSpec((1, CHUNKS, 8, 128), lambda i: (i, 0, 0, 0)),
            pl.BlockSpec((1, 8, 128), lambda i: (i, 0, 0)),
        ],
        out_specs=[
            pl.BlockSpec((1, 8, 128), lambda i: (i, 0, 0)),
            pl.BlockSpec((1, 8, 128), lambda i: (i, 0, 0)),
        ],
        out_shape=[
            jax.ShapeDtypeStruct((BEAMS, 8, 128), jnp.float32),
            jax.ShapeDtypeStruct((BEAMS, 8, 128), jnp.int32),
        ],
        scratch_shapes=[pltpu.VMEM((CHUNKS, 8, 128), jnp.float32)],
    )(lp, prev)
    cu = cu3[:, 0, :]
    idx = idx3[:, 0, :]

    tsp, sidx, dt, mi, ni, oid = pl.pallas_call(
        _merge_kernel,
        out_shape=[
            jax.ShapeDtypeStruct((2, 128), jnp.float32),
            jax.ShapeDtypeStruct((2, 128), jnp.int32),
            jax.ShapeDtypeStruct((2, 128), jnp.int32),
            jax.ShapeDtypeStruct((2, 128), jnp.int32),
            jax.ShapeDtypeStruct((1, 128), jnp.int32),
            jax.ShapeDtypeStruct((2, 128), jnp.int32),
        ],
    )(cu, idx)

    top_scores_p = tsp.reshape(TOTAL)
    sorted_index = sidx.reshape(TOTAL)
    draft_tokens = dt.reshape(TOTAL)
    mask_index = mi.reshape(TOTAL)
    topk_cs_p = top_scores_p[:TOPK]
    new_input_ids = ni.reshape(128)[:TOPK]
    out_ids = oid.reshape(TOTAL)[:TOPK]
    return (top_scores_p, draft_tokens, mask_index, topk_cs_p,
            new_input_ids, out_ids, sorted_index)
